# baseline (speedup 1.0000x reference)
import functools

import jax
import jax.numpy as jnp
from jax import lax
from jax.experimental import pallas as pl
from jax.experimental.pallas import tpu as pltpu
from jax.experimental.pallas import tpu_sc as plsc

VOC = 1_000_000
EMB = 64
B = 16384
K = 20
R = 20
KR = K + R
PAIR = 2 * EMB

NC = 2
NS = 16
NW = NC * NS
NB = B // NW
SB = 16
NSUB = NB // SB
TASKS = SB * KR
GCHUNK = 128
NG = TASKS // GCHUNK
NLANE = 16
NGRP = TASKS // NLANE


@functools.partial(
    pl.kernel,
    out_type=jax.ShapeDtypeStruct((B * KR,), jnp.float32),
    mesh=plsc.VectorSubcoreMesh(core_axis_name="c", subcore_axis_name="s"),
    compiler_params=pltpu.CompilerParams(
        needs_layout_passes=False, use_tc_tiling_on_sc=True),
    scratch_types=[
        pltpu.VMEM((NB,), jnp.int32),
        pltpu.VMEM((NB * KR,), jnp.int32),
        pltpu.VMEM((SB,), jnp.int32),
        pltpu.VMEM((TASKS,), jnp.int32),
        pltpu.VMEM((SB, PAIR), jnp.float32),
        pltpu.VMEM((TASKS, PAIR), jnp.float32),
        pltpu.VMEM((TASKS,), jnp.float32),
        pltpu.SemaphoreType.DMA,
    ],
)
def _sc_dots(center_hbm, cw_hbm, emb_hbm, lw_hbm, dots_hbm,
             cidx, widx, cpidx, wpidx, ebuf, wbuf, dbuf, sem):
    wid = lax.axis_index("s") * NC + lax.axis_index("c")
    b0 = pl.multiple_of(wid * NB, NB)
    t0 = pl.multiple_of(wid * (NB * KR), NB * KR)
    pltpu.sync_copy(center_hbm.at[pl.ds(b0, NB)], cidx)
    pltpu.sync_copy(cw_hbm.at[pl.ds(t0, NB * KR)], widx)

    lane = lax.iota(jnp.int32, NLANE)

    @pl.loop(0, NSUB)
    def _subchunk(s):
        sb0 = pl.multiple_of(s * SB, SB)
        st0 = pl.multiple_of(s * TASKS, TASKS)
        cpidx[:] = lax.shift_right_logical(cidx[pl.ds(sb0, SB)], 1)

        @pl.loop(0, NGRP)
        def _mk_pidx(i):
            o = pl.multiple_of(i * NLANE, NLANE)
            wpidx[pl.ds(o, NLANE)] = lax.shift_right_logical(
                widx[pl.ds(st0 + o, NLANE)], 1)

        copies = [pltpu.async_copy(emb_hbm.at[cpidx], ebuf, sem)]
        for q in range(NG):
            copies.append(pltpu.async_copy(
                lw_hbm.at[wpidx.at[pl.ds(q * GCHUNK, GCHUNK)]],
                wbuf.at[pl.ds(q * GCHUNK, GCHUNK)], sem))
        for c in copies:
            c.wait()

        @pl.loop(0, NGRP)
        def _per_g(g):
            go = pl.multiple_of(g * NLANE, NLANE)
            tvec = go + lane
            wh = (widx[pl.ds(st0 + go, NLANE)] & 1) * EMB
            bvec = tvec // KR
            ch = (plsc.load_gather(cidx, [sb0 + bvec]) & 1) * EMB
            acc = (plsc.load_gather(wbuf, [tvec, wh])
                   * plsc.load_gather(ebuf, [bvec, ch]))
            for j in range(1, EMB):
                acc = acc + (plsc.load_gather(wbuf, [tvec, wh + j])
                             * plsc.load_gather(ebuf, [bvec, ch + j]))
            dbuf[pl.ds(go, NLANE)] = acc

        pltpu.sync_copy(dbuf, dots_hbm.at[pl.ds(t0 + st0, TASKS)])


def _tc_loss_body(d_ref, o_ref):
    d = d_ref[...]
    col = lax.broadcasted_iota(jnp.int32, (B, KR), 1)
    act = jax.nn.sigmoid(d)
    pos = -jnp.log(act)
    neg = -jnp.log(1.0 - act + 1e-3)
    is_pos = col < K
    s_pos = jnp.sum(jnp.where(is_pos, pos, 0.0))
    s_neg = jnp.sum(jnp.where(is_pos, 0.0, neg))
    o_ref[0, 0] = s_pos / (B * K) + s_neg / (B * R)


_tc_loss = pl.pallas_call(
    _tc_loss_body,
    out_shape=jax.ShapeDtypeStruct((1, 1), jnp.float32),
    out_specs=pl.BlockSpec(memory_space=pltpu.SMEM),
)


def kernel(center, context, rand, embeddings, linear_w):
    center = center.astype(jnp.int32)
    cw = jnp.concatenate([context, rand], axis=1).astype(jnp.int32)
    dots = _sc_dots(center, cw.reshape(-1),
                    embeddings.reshape(VOC // 2, PAIR),
                    linear_w.reshape(VOC // 2, PAIR))
    loss = _tc_loss(dots.reshape(B, KR))
    return loss[0, 0]


# speedup vs baseline: 1.1804x; 1.1804x over previous
import functools

import jax
import jax.numpy as jnp
from jax import lax
from jax.experimental import pallas as pl
from jax.experimental.pallas import tpu as pltpu
from jax.experimental.pallas import tpu_sc as plsc

VOC = 1_000_000
EMB = 64
B = 16384
K = 20
R = 20
KR = K + R

NC = 2
NS = 16
NW = NC * NS
NB = B // NW
SB = 16
NSUB = NB // SB
TASKS = SB * KR
GCHUNK = 128
NG = TASKS // GCHUNK
NLANE = 16
NGRP = TASKS // NLANE

_IL = plsc.PackFormat.INTERLEAVED


@functools.partial(
    pl.kernel,
    out_type=jax.ShapeDtypeStruct((B * KR,), jnp.float32),
    mesh=plsc.VectorSubcoreMesh(core_axis_name="c", subcore_axis_name="s"),
    compiler_params=pltpu.CompilerParams(
        needs_layout_passes=False, use_tc_tiling_on_sc=False),
    scratch_types=[
        pltpu.VMEM((NB,), jnp.int32),
        pltpu.VMEM((NB * KR,), jnp.int32),
        pltpu.VMEM((SB, EMB), jnp.bfloat16),
        pltpu.VMEM((SB, EMB), jnp.bfloat16),
        pltpu.VMEM((TASKS, EMB), jnp.bfloat16),
        pltpu.VMEM((TASKS, EMB), jnp.bfloat16),
        pltpu.VMEM((TASKS * NLANE,), jnp.float32),
        pltpu.VMEM((TASKS,), jnp.float32),
        pltpu.SemaphoreType.DMA,
        pltpu.SemaphoreType.DMA,
    ],
)
def _sc_dots(center_hbm, cw_hbm, emb_hbm, lw_hbm, dots_hbm,
             cidx, widx, ebuf0, ebuf1, wbuf0, wbuf1, pbuf, dbuf, sem0, sem1):
    wid = lax.axis_index("s") * NC + lax.axis_index("c")
    b0 = pl.multiple_of(wid * NB, NB)
    t0 = pl.multiple_of(wid * (NB * KR), NB * KR)
    pltpu.sync_copy(center_hbm.at[pl.ds(b0, NB)], cidx)
    pltpu.sync_copy(cw_hbm.at[pl.ds(t0, NB * KR)], widx)

    lane = lax.iota(jnp.int32, NLANE)

    def _copies(s, ebuf, wbuf, sem):
        sb0 = pl.multiple_of(s * SB, SB)
        st0 = pl.multiple_of(s * TASKS, TASKS)
        yield pltpu.make_async_copy(
            emb_hbm.at[cidx.at[pl.ds(sb0, SB)]], ebuf, sem)
        for q in range(NG):
            yield pltpu.make_async_copy(
                lw_hbm.at[widx.at[pl.ds(st0 + q * GCHUNK, GCHUNK)]],
                wbuf.at[pl.ds(q * GCHUNK, GCHUNK)], sem)

    def _issue(s, ebuf, wbuf, sem):
        for c in _copies(s, ebuf, wbuf, sem):
            c.start()

    def _wait(s, ebuf, wbuf, sem):
        for c in _copies(s, ebuf, wbuf, sem):
            c.wait()

    def _compute(s, ebuf, wbuf):
        st0 = pl.multiple_of(s * TASKS, TASKS)

        @pl.loop(0, SB)
        def _per_b(b):
            e0e, e0o = plsc.unpack(ebuf[b, pl.ds(0, 32)], format=_IL)
            e1e, e1o = plsc.unpack(ebuf[b, pl.ds(32, 32)], format=_IL)

            @pl.loop(0, KR)
            def _per_k(k):
                t = b * KR + k
                w0e, w0o = plsc.unpack(wbuf[t, pl.ds(0, 32)], format=_IL)
                w1e, w1o = plsc.unpack(wbuf[t, pl.ds(32, 32)], format=_IL)
                p = w0e * e0e + w0o * e0o + w1e * e1e + w1o * e1o
                pbuf[pl.ds(pl.multiple_of(t * NLANE, NLANE), NLANE)] = p

        @pl.loop(0, NGRP)
        def _per_g(g):
            base = g * (NLANE * NLANE) + lane * NLANE
            acc = plsc.load_gather(pbuf, [base])
            for j in range(1, NLANE):
                acc = acc + plsc.load_gather(pbuf, [base + j])
            dbuf[pl.ds(pl.multiple_of(g * NLANE, NLANE), NLANE)] = acc

        pltpu.sync_copy(dbuf, dots_hbm.at[pl.ds(t0 + st0, TASKS)])

    _issue(0, ebuf0, wbuf0, sem0)

    @pl.loop(0, NSUB // 2)
    def _pair(h):
        s0 = h * 2
        _issue(s0 + 1, ebuf1, wbuf1, sem1)
        _wait(s0, ebuf0, wbuf0, sem0)
        _compute(s0, ebuf0, wbuf0)

        @pl.when(h < NSUB // 2 - 1)
        def _():
            _issue(s0 + 2, ebuf0, wbuf0, sem0)

        _wait(s0 + 1, ebuf1, wbuf1, sem1)
        _compute(s0 + 1, ebuf1, wbuf1)


def _tc_loss_body(d_ref, o_ref):
    d = d_ref[...]
    col = lax.broadcasted_iota(jnp.int32, (B, KR), 1)
    act = jax.nn.sigmoid(d)
    pos = -jnp.log(act)
    neg = -jnp.log(1.0 - act + 1e-3)
    is_pos = col < K
    s_pos = jnp.sum(jnp.where(is_pos, pos, 0.0))
    s_neg = jnp.sum(jnp.where(is_pos, 0.0, neg))
    o_ref[0, 0] = s_pos / (B * K) + s_neg / (B * R)


_tc_loss = pl.pallas_call(
    _tc_loss_body,
    out_shape=jax.ShapeDtypeStruct((1, 1), jnp.float32),
    out_specs=pl.BlockSpec(memory_space=pltpu.SMEM),
)


def kernel(center, context, rand, embeddings, linear_w):
    center = center.astype(jnp.int32)
    cw = jnp.concatenate([context, rand], axis=1).astype(jnp.int32)
    dots = _sc_dots(center, cw.reshape(-1),
                    embeddings.astype(jnp.bfloat16),
                    linear_w.astype(jnp.bfloat16))
    loss = _tc_loss(dots.reshape(B, KR))
    return loss[0, 0]
